# Initial kernel scaffold; baseline (speedup 1.0000x reference)
import jax
import jax.numpy as jnp
from jax.experimental import pallas as pl


def kernel(z, pos, batch, emb, mlp_w1, mlp_b1, mlp_w2, mlp_b2, cf_lin1_w, cf_lin2_w, cf_lin2_b, lin_w, lin_b, out_w, out_b):
    raise NotImplementedError("write your pallas kernel here")


# speedup vs baseline: 7.1922x; 7.1922x over previous
import functools
import numpy as np
import jax
import jax.numpy as jnp
from jax import lax
from jax.experimental import pallas as pl
from jax.experimental.pallas import tpu as pltpu
from jax.experimental.pallas import tpu_sc as plsc

H = 32
NB = 32
NG = 50
CUT = 5.0
LOG2 = float(np.log(2.0))
_OFFV = np.linspace(0.0, CUT, NG).astype(np.float32)
_CO = float(-0.5 / float(_OFFV[1] - _OFFV[0]) ** 2)
BIG = 1e30
CW = 512


def _blk(n):
    for c in (200, 250, 125, 100, 50, 40, 25, 10, 8, 5, 2, 1):
        if n % c == 0:
            return c
    return 1


def _ssp(x):
    return jnp.maximum(x, 0.0) + jnp.log1p(jnp.exp(-jnp.abs(x))) - LOG2


def _graph_body(lohi_ref, pos_ref, posT_ref, bat_ref, batT_ref,
                ew_ref, src_ref, val_ref, *, B):
    b = pl.program_id(0)
    lo = lohi_ref[b, 0]
    hi = lohi_ref[b, 1]
    px = pos_ref[:, 0:1]
    py = pos_ref[:, 1:2]
    pz = pos_ref[:, 2:3]
    bb = bat_ref[...]
    gid = lax.broadcasted_iota(jnp.int32, (B, 1), 0) + b * B
    bd0 = jnp.full((B, NB), BIG, jnp.float32)
    bi0 = jnp.zeros((B, NB), jnp.int32)
    c0 = lo // CW

    def cond(st):
        c, _, _ = st
        return c * CW < hi

    def body(st):
        c, bd, bi = st
        start = c * CW
        pcx = posT_ref[0:1, pl.ds(start, CW)]
        pcy = posT_ref[1:2, pl.ds(start, CW)]
        pcz = posT_ref[2:3, pl.ds(start, CW)]
        bc = batT_ref[0:1, pl.ds(start, CW)]
        d = jnp.sqrt((px - pcx) ** 2 + (py - pcy) ** 2 + (pz - pcz) ** 2)
        cand = start + lax.broadcasted_iota(jnp.int32, (1, CW), 1)
        ok = (bc == bb) & (cand != gid) & (d < CUT)
        dm = jnp.where(ok, d, BIG)
        dcomb0 = jnp.concatenate([bd, dm], axis=1)
        icomb = jnp.concatenate(
            [bi, jnp.broadcast_to(cand, (B, CW))], axis=1)

        def sel(k, st2):
            dc, nbd, nbi = st2
            am = jnp.argmin(dc, axis=1)
            li = lax.broadcasted_iota(jnp.int32, (B, NB + CW), 1)
            oh = li == am[:, None]
            m = jnp.min(dc, axis=1, keepdims=True)
            si = jnp.sum(jnp.where(oh, icomb, 0), axis=1, keepdims=True)
            ki = lax.broadcasted_iota(jnp.int32, (B, NB), 1)
            nbd = jnp.where(ki == k, m, nbd)
            nbi = jnp.where(ki == k, si, nbi)
            dc = jnp.where(oh, BIG, dc)
            return dc, nbd, nbi

        _, bd, bi = lax.fori_loop(0, NB, sel, (dcomb0, bd, bi))
        return c + 1, bd, bi

    _, bd, bi = lax.while_loop(cond, body, (c0, bd0, bi0))
    okf = bd < (BIG * 0.5)
    ew_ref[...] = jnp.where(okf, bd, 0.0)
    src_ref[...] = bi
    val_ref[...] = okf.astype(jnp.float32)


def _build_graph(pos, batch, B):
    n = pos.shape[0]
    g = n // B
    np2 = ((n + CW - 1) // CW) * CW
    posT = jnp.pad(pos.T, ((0, 0), (0, np2 - n)))
    batf = batch.astype(jnp.float32)
    batT = jnp.pad(batf[None, :], ((0, 0), (0, np2 - n)),
                   constant_values=-1.0)
    batv = batf[:, None]
    b2 = batch.reshape(g, B)
    lo = jnp.searchsorted(batch, b2[:, 0], side='left').astype(jnp.int32)
    hi = jnp.searchsorted(batch, b2[:, -1], side='right').astype(jnp.int32)
    lohi = jnp.stack([lo, hi], axis=1)

    grid_spec = pltpu.PrefetchScalarGridSpec(
        num_scalar_prefetch=1,
        grid=(g,),
        in_specs=[
            pl.BlockSpec((B, 3), lambda i, s: (i, 0)),
            pl.BlockSpec((3, np2), lambda i, s: (0, 0)),
            pl.BlockSpec((B, 1), lambda i, s: (i, 0)),
            pl.BlockSpec((1, np2), lambda i, s: (0, 0)),
        ],
        out_specs=[
            pl.BlockSpec((B, NB), lambda i, s: (i, 0)),
            pl.BlockSpec((B, NB), lambda i, s: (i, 0)),
            pl.BlockSpec((B, NB), lambda i, s: (i, 0)),
        ],
    )
    ew, src, val = pl.pallas_call(
        functools.partial(_graph_body, B=B),
        grid_spec=grid_spec,
        out_shape=[
            jax.ShapeDtypeStruct((n, NB), jnp.float32),
            jax.ShapeDtypeStruct((n, NB), jnp.int32),
            jax.ShapeDtypeStruct((n, NB), jnp.float32),
        ],
    )(lohi, pos, posT, batv, batT)
    return ew, src, val


def _embed_body(z_ref, emb_ref, cf1_ref, h_ref, xf_ref, *, B, zmax):
    zc = z_ref[...]
    zi = lax.broadcasted_iota(jnp.int32, (1, zmax), 1)
    oh = (zc == zi).astype(jnp.float32)
    h = jnp.dot(oh, emb_ref[...], preferred_element_type=jnp.float32)
    h_ref[...] = h
    xf_ref[...] = jnp.dot(h, cf1_ref[...], preferred_element_type=jnp.float32)


def _embed(z, emb, cf1, B):
    n = z.shape[0]
    g = n // B
    zmax = emb.shape[0]
    zc = z.astype(jnp.int32)[:, None]
    return pl.pallas_call(
        functools.partial(_embed_body, B=B, zmax=zmax),
        grid=(g,),
        in_specs=[
            pl.BlockSpec((B, 1), lambda i: (i, 0)),
            pl.BlockSpec((zmax, H), lambda i: (0, 0)),
            pl.BlockSpec((H, H), lambda i: (0, 0)),
        ],
        out_specs=[
            pl.BlockSpec((B, H), lambda i: (i, 0)),
            pl.BlockSpec((B, H), lambda i: (i, 0)),
        ],
        out_shape=[
            jax.ShapeDtypeStruct((n, H), jnp.float32),
            jax.ShapeDtypeStruct((n, H), jnp.float32),
        ],
    )(zc, emb, cf1)


def _layer_body(h_ref, xs_ref, ew_ref, val_ref, mw1_ref, mb1_ref, mw2_ref,
                mb2_ref, cf2_ref, cb2_ref, lw_ref, lb_ref, cf1n_ref,
                h_out, xf_out, *, B):
    ew = ew_ref[...]
    off = lax.broadcasted_iota(jnp.int32, (1, NG), 1).astype(jnp.float32) \
        * (CUT / (NG - 1))
    ea = jnp.exp(_CO * (ew - off) ** 2)
    t = _ssp(jnp.dot(ea, mw1_ref[...], preferred_element_type=jnp.float32)
             + mb1_ref[...])
    w = jnp.dot(t, mw2_ref[...], preferred_element_type=jnp.float32) \
        + mb2_ref[...]
    cc = 0.5 * (jnp.cos(ew * (np.pi / CUT)) + 1.0)
    w = w * cc
    msg = jnp.where(val_ref[...] > 0.0, xs_ref[:, :H] * w, 0.0)
    agg = jnp.sum(msg.reshape(B, NB, H), axis=1)
    v = _ssp(jnp.dot(agg, cf2_ref[...], preferred_element_type=jnp.float32)
             + cb2_ref[...])
    v = jnp.dot(v, lw_ref[...], preferred_element_type=jnp.float32) \
        + lb_ref[...]
    hn = h_ref[...] + v
    h_out[...] = hn
    xf_out[...] = jnp.dot(hn, cf1n_ref[...],
                          preferred_element_type=jnp.float32)


def _layer(h, xs, ew_e, val_e, mw1, mb1, mw2, mb2, cf2, cb2, lw, lb, cf1n, B):
    n = h.shape[0]
    g = n // B
    eb = B * NB
    wspec = lambda shp: pl.BlockSpec(shp, lambda i: (0, 0))
    return pl.pallas_call(
        functools.partial(_layer_body, B=B),
        grid=(g,),
        in_specs=[
            pl.BlockSpec((B, H), lambda i: (i, 0)),
            pl.BlockSpec((eb, 128), lambda i: (i, 0)),
            pl.BlockSpec((eb, 1), lambda i: (i, 0)),
            pl.BlockSpec((eb, 1), lambda i: (i, 0)),
            wspec((NG, H)), wspec((1, H)), wspec((H, H)), wspec((1, H)),
            wspec((H, H)), wspec((1, H)), wspec((H, H)), wspec((1, H)),
            wspec((H, H)),
        ],
        out_specs=[
            pl.BlockSpec((B, H), lambda i: (i, 0)),
            pl.BlockSpec((B, H), lambda i: (i, 0)),
        ],
        out_shape=[
            jax.ShapeDtypeStruct((n, H), jnp.float32),
            jax.ShapeDtypeStruct((n, H), jnp.float32),
        ],
    )(h, xs, ew_e, val_e, mw1, mb1[None, :], mw2, mb2[None, :],
      cf2, cb2[None, :], lw, lb[None, :], cf1n)


def _out_body(h_ref, ow_ref, ob_ref, o_ref):
    o_ref[...] = _ssp(
        jnp.dot(h_ref[...], ow_ref[...], preferred_element_type=jnp.float32)
        + ob_ref[...])


def _outproj(h, ow, ob, B):
    n = h.shape[0]
    g = n // B
    ho = ow.shape[1]
    return pl.pallas_call(
        _out_body,
        grid=(g,),
        in_specs=[
            pl.BlockSpec((B, H), lambda i: (i, 0)),
            pl.BlockSpec((H, ho), lambda i: (0, 0)),
            pl.BlockSpec((1, ho), lambda i: (0, 0)),
        ],
        out_specs=pl.BlockSpec((B, ho), lambda i: (i, 0)),
        out_shape=jax.ShapeDtypeStruct((n, ho), jnp.float32),
    )(h, ow, ob[None, :])


_SC_CHUNK = 128


def _sc_gather_call(xf, idx2d, nw, nc, ns, per_w):
    nch = per_w // _SC_CHUNK
    mesh = plsc.VectorSubcoreMesh(core_axis_name="c", subcore_axis_name="s",
                                  num_cores=nc, num_subcores=ns)

    def body(xf_hbm, idx_hbm, out_hbm, idx_v, rows_v, sem):
        wid = lax.axis_index("s") * nc + lax.axis_index("c")

        @pl.loop(0, nch)
        def _(i):
            base = wid * per_w + i * _SC_CHUNK
            pltpu.sync_copy(idx_hbm.at[wid, pl.ds(i * _SC_CHUNK, _SC_CHUNK)],
                            idx_v)
            pltpu.async_copy(xf_hbm.at[idx_v], rows_v, sem).wait()
            pltpu.sync_copy(rows_v, out_hbm.at[pl.ds(base, _SC_CHUNK)])

    return pl.kernel(
        body,
        out_type=jax.ShapeDtypeStruct((nw * per_w, 128), jnp.float32),
        mesh=mesh,
        scratch_types=[
            pltpu.VMEM((_SC_CHUNK,), jnp.int32),
            pltpu.VMEM((_SC_CHUNK, 128), jnp.float32),
            pltpu.SemaphoreType.DMA,
        ],
    )(xf, idx2d)


def _gather_rows(xf, srcf):
    e = srcf.shape[0]
    nc, ns = 2, 16
    nw = nc * ns
    gran = nw * _SC_CHUNK
    e_pad = ((e + gran - 1) // gran) * gran
    per_w = e_pad // nw
    xfp = jnp.pad(xf, ((0, 0), (0, 128 - H)))
    idx2d = jnp.pad(srcf.astype(jnp.int32), (0, e_pad - e)).reshape(nw, per_w)
    return _sc_gather_call(xfp, idx2d, nw, nc, ns, per_w)


def kernel(z, pos, batch, emb, mlp_w1, mlp_b1, mlp_w2, mlp_b2,
           cf_lin1_w, cf_lin2_w, cf_lin2_b, lin_w, lin_b, out_w, out_b):
    n = pos.shape[0]
    B = _blk(n)
    L = mlp_w1.shape[0]
    ew, src, val = _build_graph(pos, batch.astype(jnp.int32), B)
    ew_e = ew.reshape(n * NB, 1)
    val_e = val.reshape(n * NB, 1)
    srcf = src.reshape(n * NB)
    h, xf = _embed(z, emb, cf_lin1_w[0], B)
    for l in range(L):
        xs = _gather_rows(xf, srcf)
        cf1n = cf_lin1_w[min(l + 1, L - 1)]
        h, xf = _layer(h, xs, ew_e, val_e, mlp_w1[l], mlp_b1[l], mlp_w2[l],
                       mlp_b2[l], cf_lin2_w[l], cf_lin2_b[l], lin_w[l],
                       lin_b[l], cf1n, B)
    return _outproj(h, out_w, out_b, B)
